# baseline (speedup 1.0000x reference)
import functools

import jax
import jax.numpy as jnp
from jax import lax
from jax.experimental import pallas as pl
from jax.experimental.pallas import tpu as pltpu
from jax.experimental.pallas import tpu_sc as plsc

B = 4
C_IN = 512
C_OUT = 384
D = 8192
SPLIT = 4
DSUB = D // SPLIT
NROWS_OUT = B * C_OUT * SPLIT
NW = 32
ROWS_PER_TILE = NROWS_OUT // NW
CHUNK = 24
NCHUNK = ROWS_PER_TILE // CHUNK

_mesh = plsc.VectorSubcoreMesh(core_axis_name="c", subcore_axis_name="s")


@functools.partial(
    pl.kernel,
    mesh=_mesh,
    out_type=jax.ShapeDtypeStruct((NROWS_OUT, DSUB), jnp.float32),
    scratch_types=[
        pltpu.VMEM((NCHUNK, CHUNK), jnp.int32),
        pltpu.VMEM((CHUNK, DSUB), jnp.float32),
        pltpu.VMEM((CHUNK, DSUB), jnp.float32),
        pltpu.SemaphoreType.DMA,
        pltpu.SemaphoreType.DMA,
        pltpu.SemaphoreType.DMA,
        pltpu.SemaphoreType.DMA,
    ],
)
def _sc_gather(table_hbm, fidx_hbm, out_hbm, idx_v, buf0, buf1,
               gsem0, gsem1, ssem0, ssem1):
    wid = lax.axis_index("s") * 2 + lax.axis_index("c")
    base = wid * ROWS_PER_TILE
    pltpu.sync_copy(fidx_hbm.at[wid], idx_v)

    bufs = (buf0, buf1)
    gsems = (gsem0, gsem1)
    ssems = (ssem0, ssem1)
    gathers = [None, None]
    scatters = [None, None]

    gathers[0] = pltpu.async_copy(table_hbm.at[idx_v.at[0]], buf0, gsem0)
    for c in range(NCHUNK):
        nxt = c + 1
        if nxt < NCHUNK:
            s = nxt % 2
            if scatters[s] is not None:
                scatters[s].wait()
                scatters[s] = None
            gathers[s] = pltpu.async_copy(
                table_hbm.at[idx_v.at[nxt]], bufs[s], gsems[s])
        cur = c % 2
        gathers[cur].wait()
        scatters[cur] = pltpu.async_copy(
            bufs[cur], out_hbm.at[pl.ds(base + c * CHUNK, CHUNK)], ssems[cur])
    for s in range(2):
        if scatters[s] is not None:
            scatters[s].wait()


def kernel(x, indices):
    table = x.reshape(B * C_IN * SPLIT, DSUB)
    rows = jnp.arange(B, dtype=jnp.int32)[:, None] * C_IN + indices[None, :]
    fidx = (rows[:, :, None] * SPLIT
            + jnp.arange(SPLIT, dtype=jnp.int32)[None, None, :])
    fidx = fidx.reshape(NW, NCHUNK, CHUNK)
    out = _sc_gather(table, fidx)
    return out.reshape(B, C_OUT, D)


# speedup vs baseline: 3.1475x; 3.1475x over previous
import functools

import jax
import jax.numpy as jnp
from jax import lax
from jax.experimental import pallas as pl
from jax.experimental.pallas import tpu as pltpu
from jax.experimental.pallas import tpu_sc as plsc

B = 4
C_IN = 512
C_OUT = 384
D = 8192
NROWS_OUT = B * C_OUT
NW = 32
ROWS_PER_TILE = NROWS_OUT // NW
CHUNK = 8
NCHUNK = ROWS_PER_TILE // CHUNK

_mesh = plsc.VectorSubcoreMesh(core_axis_name="c", subcore_axis_name="s")


@functools.partial(
    pl.kernel,
    mesh=_mesh,
    out_type=jax.ShapeDtypeStruct((NROWS_OUT, D), jnp.float32),
    scratch_types=[
        pltpu.VMEM((NCHUNK, CHUNK), jnp.int32),
        pltpu.VMEM((CHUNK, D), jnp.float32),
        pltpu.SemaphoreType.DMA,
        pltpu.SemaphoreType.DMA,
    ],
)
def _sc_gather(table_hbm, fidx_hbm, out_hbm, idx_v, buf, gsem, ssem):
    wid = lax.axis_index("s") * 2 + lax.axis_index("c")
    base = wid * ROWS_PER_TILE
    pltpu.sync_copy(fidx_hbm.at[wid], idx_v)

    scatter = None
    for c in range(NCHUNK):
        if scatter is not None:
            scatter.wait()
        g = pltpu.async_copy(table_hbm.at[idx_v.at[c]], buf, gsem)
        g.wait()
        scatter = pltpu.async_copy(
            buf, out_hbm.at[pl.ds(base + c * CHUNK, CHUNK)], ssem)
    scatter.wait()


def kernel(x, indices):
    table = x.reshape(B * C_IN, D)
    fidx = jnp.arange(B, dtype=jnp.int32)[:, None] * C_IN + indices[None, :]
    fidx = fidx.reshape(NW, NCHUNK, CHUNK)
    out = _sc_gather(table, fidx)
    return out.reshape(B, C_OUT, D)


# speedup vs baseline: 3.2764x; 1.0410x over previous
import functools

import jax
import jax.numpy as jnp
from jax import lax
from jax.experimental import pallas as pl
from jax.experimental.pallas import tpu as pltpu
from jax.experimental.pallas import tpu_sc as plsc

B = 4
C_IN = 512
C_OUT = 384
D = 8192
NROWS_OUT = B * C_OUT
NW = 32
ROWS_PER_TILE = NROWS_OUT // NW
CHUNK = 4
NCHUNK = ROWS_PER_TILE // CHUNK

_mesh = plsc.VectorSubcoreMesh(core_axis_name="c", subcore_axis_name="s")


@functools.partial(
    pl.kernel,
    mesh=_mesh,
    out_type=jax.ShapeDtypeStruct((NROWS_OUT, D), jnp.float32),
    scratch_types=[
        pltpu.VMEM((NCHUNK, CHUNK), jnp.int32),
        pltpu.VMEM((CHUNK, D), jnp.float32),
        pltpu.VMEM((CHUNK, D), jnp.float32),
        pltpu.SemaphoreType.DMA,
        pltpu.SemaphoreType.DMA,
        pltpu.SemaphoreType.DMA,
        pltpu.SemaphoreType.DMA,
    ],
)
def _sc_gather(table_hbm, fidx_hbm, out_hbm, idx_v, buf0, buf1,
               gsem0, gsem1, ssem0, ssem1):
    wid = lax.axis_index("s") * 2 + lax.axis_index("c")
    base = wid * ROWS_PER_TILE
    pltpu.sync_copy(fidx_hbm.at[wid], idx_v)

    bufs = (buf0, buf1)
    gsems = (gsem0, gsem1)
    ssems = (ssem0, ssem1)
    gathers = [None, None]
    scatters = [None, None]

    gathers[0] = pltpu.async_copy(table_hbm.at[idx_v.at[0]], buf0, gsem0)
    for c in range(NCHUNK):
        nxt = c + 1
        if nxt < NCHUNK:
            s = nxt % 2
            if scatters[s] is not None:
                scatters[s].wait()
                scatters[s] = None
            gathers[s] = pltpu.async_copy(
                table_hbm.at[idx_v.at[nxt]], bufs[s], gsems[s])
        cur = c % 2
        gathers[cur].wait()
        scatters[cur] = pltpu.async_copy(
            bufs[cur], out_hbm.at[pl.ds(base + c * CHUNK, CHUNK)], ssems[cur])
    for s in range(2):
        if scatters[s] is not None:
            scatters[s].wait()


def kernel(x, indices):
    table = x.reshape(B * C_IN, D)
    fidx = jnp.arange(B, dtype=jnp.int32)[:, None] * C_IN + indices[None, :]
    fidx = fidx.reshape(NW, NCHUNK, CHUNK)
    out = _sc_gather(table, fidx)
    return out.reshape(B, C_OUT, D)
